# Initial kernel scaffold; baseline (speedup 1.0000x reference)
import jax
import jax.numpy as jnp
from jax.experimental import pallas as pl


def kernel(x, codebook):
    raise NotImplementedError("write your pallas kernel here")


# speedup vs baseline: 2.9924x; 2.9924x over previous
import jax
import jax.numpy as jnp
from jax.experimental import pallas as pl
from jax.experimental.pallas import tpu as pltpu

_NB = 1024
_D = 256
_EPS = 1e-07


def _vq_kernel(x_ref, cb_ref, xout_ref, idx_ref, commit_ref, ppl_ref,
               count_acc, commit_acc):
    i = pl.program_id(0)
    n = pl.num_programs(0)
    xblk = x_ref[0]
    cb = cb_ref[...]

    mm = jax.lax.dot_general(xblk, cb, (((0,), (1,)), ((), ())),
                             preferred_element_type=jnp.float32)
    xnorm = jnp.sum(xblk * xblk, axis=0)
    cnorm = jnp.sum(cb * cb, axis=1)
    dist = (xnorm[:, None] - 2.0 * mm) + cnorm[None, :]

    minval = jnp.min(dist, axis=1)
    lane = jax.lax.broadcasted_iota(jnp.int32, dist.shape, 1)
    idx = jnp.min(jnp.where(dist == minval[:, None], lane, _NB),
                  axis=1).astype(jnp.int32)
    idx_ref[0, 0, :] = idx

    onehot = (lane == idx[:, None]).astype(jnp.float32)
    xo = jax.lax.dot_general(cb, onehot, (((0,), (1,)), ((), ())),
                             preferred_element_type=jnp.float32)
    xout_ref[0] = xblk + (xo - xblk)

    diff = xblk - xo
    part_commit = jnp.sum(diff * diff)
    part_count = jnp.sum(onehot, axis=0)[None, :]

    @pl.when(i == 0)
    def _init():
        count_acc[...] = part_count
        commit_acc[0, 0] = part_commit

    @pl.when(i > 0)
    def _acc():
        count_acc[...] = count_acc[...] + part_count
        commit_acc[0, 0] = commit_acc[0, 0] + part_commit

    @pl.when(i == n - 1)
    def _final():
        counts = count_acc[...]
        total = jnp.sum(counts)
        prob = counts / total
        ppl = jnp.exp(-jnp.sum(prob * jnp.log(prob + _EPS)))
        ppl_ref[0, 0] = ppl
        commit_ref[0, 0] = commit_acc[0, 0] / (total * _D)


def kernel(x, codebook):
    N, D, T = x.shape
    grid = (N,)
    out_shapes = (
        jax.ShapeDtypeStruct((N, D, T), jnp.float32),
        jax.ShapeDtypeStruct((N, 1, T), jnp.int32),
        jax.ShapeDtypeStruct((1, 1), jnp.float32),
        jax.ShapeDtypeStruct((1, 1), jnp.float32),
    )
    x_out, idx, commit, ppl = pl.pallas_call(
        _vq_kernel,
        grid=grid,
        in_specs=[
            pl.BlockSpec((1, D, T), lambda i: (i, 0, 0)),
            pl.BlockSpec((_NB, _D), lambda i: (0, 0)),
        ],
        out_specs=(
            pl.BlockSpec((1, D, T), lambda i: (i, 0, 0)),
            pl.BlockSpec((1, 1, T), lambda i: (i, 0, 0)),
            pl.BlockSpec(memory_space=pltpu.SMEM),
            pl.BlockSpec(memory_space=pltpu.SMEM),
        ),
        out_shape=out_shapes,
        scratch_shapes=[
            pltpu.VMEM((1, _NB), jnp.float32),
            pltpu.SMEM((1, 1), jnp.float32),
        ],
    )(x, codebook)
    return (x_out,
            idx.reshape(N, T),
            commit.reshape(()),
            ppl.reshape(()))
